# Initial kernel scaffold; baseline (speedup 1.0000x reference)
import jax
import jax.numpy as jnp
from jax.experimental import pallas as pl


def kernel(users, items, user_emb, item_emb, rating_emb, W1, b1, W2, b2, edge_src, edge_dst, edge_val):
    raise NotImplementedError("write your pallas kernel here")


# speedup vs baseline: 1.0603x; 1.0603x over previous
import jax
import jax.numpy as jnp
from jax.experimental import pallas as pl

_NU = 50000
_NI = 50000
_N = _NU + _NI
_D = 64
_R = 5
_NL = 2
_BN = 1000


def _mlp_block(base_ref, remb_ref, w1_ref, b1_ref, w2_ref, b2_ref, out_ref):
    r = pl.program_id(0)
    base = base_ref[...]
    w1 = w1_ref[0]
    w1_top = w1[:_D, :]
    w1_bot = w1[_D:, :]
    remb = remb_ref[pl.ds(r + 1, 1)]
    rvec = jnp.dot(remb, w1_bot, preferred_element_type=jnp.float32)
    z = jnp.dot(base, w1_top, preferred_element_type=jnp.float32) + rvec + b1_ref[pl.ds(r, 1)]
    z = jnp.where(z >= 0, z, 0.01 * z)
    h = jnp.dot(z, w2_ref[0], preferred_element_type=jnp.float32) + b2_ref[pl.ds(r, 1)]
    out_ref[0] = h


def kernel(users, items, user_emb, item_emb, rating_emb, W1, b1, W2, b2, edge_src, edge_dst, edge_val):
    all_base = jnp.concatenate([user_emb, item_emb], axis=0)

    h_all = pl.pallas_call(
        _mlp_block,
        grid=(_R, _N // _BN),
        in_specs=[
            pl.BlockSpec((_BN, _D), lambda r, n: (n, 0)),
            pl.BlockSpec((_R + 1, _D), lambda r, n: (0, 0)),
            pl.BlockSpec((1, 2 * _D, _D), lambda r, n: (r, 0, 0)),
            pl.BlockSpec((_R, _D), lambda r, n: (0, 0)),
            pl.BlockSpec((1, _D, _D), lambda r, n: (r, 0, 0)),
            pl.BlockSpec((_R, _D), lambda r, n: (0, 0)),
        ],
        out_specs=pl.BlockSpec((1, _BN, _D), lambda r, n: (r, n, 0)),
        out_shape=jax.ShapeDtypeStruct((_R, _N, _D), jnp.float32),
    )(all_base, rating_emb, W1, b1, W2, b2)

    def prop(h, src, dst, val):
        total = h
        cur = h
        for _ in range(_NL):
            msg = val[:, None] * cur[src]
            cur = jax.ops.segment_sum(msg, dst, num_segments=_N)
            total = total + cur
        return total

    outs = jax.vmap(prop)(h_all, edge_src, edge_dst, edge_val)
    out = jnp.mean(outs, axis=0)
    users_emb = out[:_NU][users]
    pos_emb = out[_NU:][items]
    return jnp.sum(users_emb * pos_emb, axis=1)
